# Initial kernel scaffold; baseline (speedup 1.0000x reference)
import jax
import jax.numpy as jnp
from jax.experimental import pallas as pl


def kernel(x, edge_index, edge_attr, params):
    raise NotImplementedError("write your pallas kernel here")


# speedup vs baseline: 1.2200x; 1.2200x over previous
import dataclasses
import functools

import jax
import jax.numpy as jnp
from jax import lax
from jax.experimental import pallas as pl
from jax.experimental.pallas import tpu as pltpu
from jax.experimental.pallas import tpu_sc as plsc

N = 10000
E = 160000
H = 256
HEADS = 8
DH = H // HEADS
NC = 2
NS = 16
LANES = 16
HHALF = H // NC
HPC = HEADS // NC
VPH = DH // LANES

EPB = 48
E_PER_SUB = 10032
E_PAD = E_PER_SUB * NS
NP = 10112
NODES_PER_SUB = NP // NS
ZBLK = 8
INV_SQRT_DH = 1.0 / (DH ** 0.5)

BN = 400
BE = 640


def _proj_body(x_ref, wq_ref, wk_ref, wv_ref, bq_ref, q_ref, k_ref, v_ref):
    xb = x_ref[...]
    q_ref[...] = jnp.dot(xb, wq_ref[...], preferred_element_type=jnp.float32) + bq_ref[0]
    k_ref[...] = jnp.dot(xb, wk_ref[...], preferred_element_type=jnp.float32)
    v_ref[...] = jnp.dot(xb, wv_ref[...], preferred_element_type=jnp.float32)


def _project_nodes(x, wqt, wkt, wvt, bq2):
    out = jax.ShapeDtypeStruct((N, H), jnp.float32)
    full = lambda shp: pl.BlockSpec(shp, lambda i: tuple(0 for _ in shp))
    return pl.pallas_call(
        _proj_body,
        grid=(N // BN,),
        in_specs=[
            pl.BlockSpec((BN, H), lambda i: (i, 0)),
            full((H, H)), full((H, H)), full((H, H)), full((1, H)),
        ],
        out_specs=[
            pl.BlockSpec((BN, H), lambda i: (i, 0)),
            pl.BlockSpec((BN, H), lambda i: (i, 0)),
            pl.BlockSpec((BN, H), lambda i: (i, 0)),
        ],
        out_shape=[out, out, out],
    )(x, wqt, wkt, wvt, bq2)


def _eproj_body(ea_ref, wk_ref, wv_ref, bk_ref, bv_ref, ek_ref, ev_ref):
    eb = ea_ref[...]
    ek_ref[...] = jnp.dot(eb, wk_ref[...], preferred_element_type=jnp.float32) + bk_ref[0]
    ev_ref[...] = jnp.dot(eb, wv_ref[...], preferred_element_type=jnp.float32) + bv_ref[0]


def _project_edges(edge_attr, wkt, wvt, bk2, bv2):
    out = jax.ShapeDtypeStruct((E, H), jnp.float32)
    full = lambda shp: pl.BlockSpec(shp, lambda i: tuple(0 for _ in shp))
    return pl.pallas_call(
        _eproj_body,
        grid=(E // BE,),
        in_specs=[
            pl.BlockSpec((BE, H), lambda i: (i, 0)),
            full((H, H)), full((H, H)), full((1, H)), full((1, H)),
        ],
        out_specs=[
            pl.BlockSpec((BE, H), lambda i: (i, 0)),
            pl.BlockSpec((BE, H), lambda i: (i, 0)),
        ],
        out_shape=[out, out],
    )(edge_attr, wkt, wvt, bk2, bv2)


def _edge_math_body(qg_ref, kg_ref, vg_ref, ekr_ref, evr_ref, expt_ref,
                    expand_ref, msg_ref, pe_ref):
    k = kg_ref[...] + ekr_ref[...]
    v = vg_ref[...] + evr_ref[...]
    alpha = jnp.dot(qg_ref[...] * k, expt_ref[...],
                    preferred_element_type=jnp.float32) * INV_SQRT_DH
    pe = jnp.exp(alpha)
    pe_ref[...] = pe
    msg_ref[...] = jnp.dot(pe, expand_ref[...],
                           preferred_element_type=jnp.float32) * v


def _edge_math(qg, kg, vg, ek, ev, expt, expand):
    full = lambda shp: pl.BlockSpec(shp, lambda i: tuple(0 for _ in shp))
    eb = lambda: pl.BlockSpec((BE, H), lambda i: (i, 0))
    return pl.pallas_call(
        _edge_math_body,
        grid=(E // BE,),
        in_specs=[eb(), eb(), eb(), eb(), eb(), full((H, HEADS)),
                  full((HEADS, H))],
        out_specs=[eb(), pl.BlockSpec((BE, HEADS), lambda i: (i, 0))],
        out_shape=[jax.ShapeDtypeStruct((E, H), jnp.float32),
                   jax.ShapeDtypeStruct((E, HEADS), jnp.float32)],
    )(qg, kg, vg, ek, ev, expt, expand)


def _sc_edge_body(qn_hbm, kn_hbm, vn_hbm, ek_hbm, ev_hbm, src_hbm, dst_hbm,
                  z128_hbm, z16_hbm,
                  aggr_hbm, asum_hbm,
                  idx_s, idx_d, idx_dg,
                  qv, kv, vv, ekv, evv, pbuf,
                  aggr_sh, asum_sh):
    msg = qv
    c = lax.axis_index("c")
    s = lax.axis_index("s")

    pltpu.sync_copy(z128_hbm, aggr_sh)
    pltpu.sync_copy(z16_hbm, asum_sh)

    plsc.subcore_barrier()

    ebase = s * E_PER_SUB
    noff = c * N
    eoff = c * E_PAD + ebase

    @pl.loop(0, E_PER_SUB, step=EPB)
    def _block(off):
        base = ebase + off
        pltpu.sync_copy(src_hbm.at[pl.ds(base, EPB)], idx_s)
        pltpu.sync_copy(dst_hbm.at[pl.ds(base, EPB)], idx_d)
        for t in range(EPB // LANES):
            sl = pl.ds(t * LANES, LANES)
            idx_dg[sl] = jnp.minimum(idx_d[sl], N - 1) + noff
            idx_s[sl] = idx_s[sl] + noff
        pltpu.sync_copy(qn_hbm.at[idx_dg], qv)
        pltpu.sync_copy(kn_hbm.at[idx_s], kv)
        pltpu.sync_copy(vn_hbm.at[idx_s], vv)
        pltpu.sync_copy(ek_hbm.at[pl.ds(eoff + off, EPB)], ekv)
        pltpu.sync_copy(ev_hbm.at[pl.ds(eoff + off, EPB)], evv)

        lane = lax.iota(jnp.int32, LANES)

        @pl.loop(0, EPB)
        def _edge(e):
            ps = []
            for hh in range(HPC):
                j0 = (hh * VPH) * LANES
                j1 = (hh * VPH + 1) * LANES
                k0 = kv[e, pl.ds(j0, LANES)] + ekv[e, pl.ds(j0, LANES)]
                k1 = kv[e, pl.ds(j1, LANES)] + ekv[e, pl.ds(j1, LANES)]
                q0 = qv[e, pl.ds(j0, LANES)]
                q1 = qv[e, pl.ds(j1, LANES)]
                a = (jnp.sum(q0 * k0) + jnp.sum(q1 * k1)) * INV_SQRT_DH
                p = jnp.exp(jnp.full((LANES,), a, jnp.float32))
                ps.append(p)
                v0 = vv[e, pl.ds(j0, LANES)] + evv[e, pl.ds(j0, LANES)]
                v1 = vv[e, pl.ds(j1, LANES)] + evv[e, pl.ds(j1, LANES)]
                msg[e, pl.ds(j0, LANES)] = p * v0
                msg[e, pl.ds(j1, LANES)] = p * v1
            pv = jnp.where(lane == 0, ps[0],
                           jnp.where(lane == 1, ps[1],
                                     jnp.where(lane == 2, ps[2],
                                               jnp.where(lane == 3, ps[3], 0.0))))
            pbuf[e, pl.ds(0, LANES)] = pv

        pltpu.sync_copy(msg, aggr_sh.at[idx_d], add=True)
        pltpu.sync_copy(pbuf, asum_sh.at[idx_d], add=True)

    plsc.subcore_barrier()

    pltpu.sync_copy(aggr_sh, aggr_hbm.at[c])
    pltpu.sync_copy(asum_sh, asum_hbm.at[c])


def _sc_edge_pass(qn, kn, vn, ek, ev, src, dst):
    mesh = plsc.VectorSubcoreMesh(core_axis_name="c", subcore_axis_name="s")
    cp = pltpu.CompilerParams()
    if "needs_layout_passes" in pltpu.CompilerParams.__dataclass_fields__:
        cp = dataclasses.replace(cp, needs_layout_passes=False)
    fn = pl.kernel(
        _sc_edge_body,
        mesh=mesh,
        compiler_params=cp,
        out_type=[
            jax.ShapeDtypeStruct((NC, NP, HHALF), jnp.float32),
            jax.ShapeDtypeStruct((NC, NP, LANES), jnp.float32),
        ],
        scratch_types=[
            pltpu.VMEM((EPB,), jnp.int32),
            pltpu.VMEM((EPB,), jnp.int32),
            pltpu.VMEM((EPB,), jnp.int32),
            pltpu.VMEM((EPB, HHALF), jnp.float32),
            pltpu.VMEM((EPB, HHALF), jnp.float32),
            pltpu.VMEM((EPB, HHALF), jnp.float32),
            pltpu.VMEM((EPB, HHALF), jnp.float32),
            pltpu.VMEM((EPB, HHALF), jnp.float32),
            pltpu.VMEM((EPB, LANES), jnp.float32),
            pltpu.VMEM_SHARED((NP, HHALF), jnp.float32),
            pltpu.VMEM_SHARED((NP, LANES), jnp.float32),
        ],
    )
    z128 = jnp.zeros((NP, HHALF), jnp.float32)
    z16 = jnp.zeros((NP, LANES), jnp.float32)
    return fn(qn, kn, vn, ek, ev, src, dst, z128, z16)


def _ln(v, w, b, eps=1e-12):
    u = jnp.mean(v, axis=-1, keepdims=True)
    sv = jnp.mean((v - u) ** 2, axis=-1, keepdims=True)
    return w * ((v - u) / jnp.sqrt(sv + eps)) + b


def _dense_body(aggr_ref, asum_ref, x_ref, h_ref, expand_ref,
                wao_ref, bao_ref, ln1w_ref, ln1b_ref,
                wi_ref, bi_ref, wo_ref, bo_ref, ln2w_ref, ln2b_ref,
                wih_ref, bih_ref, whh_ref, bhh_ref, ln3w_ref, ln3b_ref,
                xn_ref, hn_ref):
    x = x_ref[...]
    h = h_ref[...]
    den = jnp.dot(asum_ref[...], expand_ref[...],
                  preferred_element_type=jnp.float32) + 1e-16
    aggr = aggr_ref[...] / den
    ao = _ln(jnp.dot(aggr, wao_ref[...], preferred_element_type=jnp.float32)
             + bao_ref[0] + x, ln1w_ref[0], ln1b_ref[0])
    inter = jnp.dot(ao, wi_ref[...], preferred_element_type=jnp.float32) + bi_ref[0]
    inter = inter * 0.5 * (1.0 + lax.erf(inter / 1.41421))
    m = _ln(jnp.dot(inter, wo_ref[...], preferred_element_type=jnp.float32)
            + bo_ref[0] + ao, ln2w_ref[0], ln2b_ref[0])
    gi = jnp.dot(m, wih_ref[...], preferred_element_type=jnp.float32) + bih_ref[0]
    gh = jnp.dot(h, whh_ref[...], preferred_element_type=jnp.float32) + bhh_ref[0]
    r = jax.nn.sigmoid(gi[:, :H] + gh[:, :H])
    z = jax.nn.sigmoid(gi[:, H:2 * H] + gh[:, H:2 * H])
    n = jnp.tanh(gi[:, 2 * H:] + r * gh[:, 2 * H:])
    hn = (1.0 - z) * n + z * h
    hn_ref[...] = hn
    xn_ref[...] = _ln(hn, ln3w_ref[0], ln3b_ref[0])


def _dense_update(aggr, asum8, x, h, expand, wp):
    full = lambda shp: pl.BlockSpec(shp, lambda i: tuple(0 for _ in shp))
    out = jax.ShapeDtypeStruct((N, H), jnp.float32)
    return pl.pallas_call(
        _dense_body,
        grid=(N // BN,),
        in_specs=[
            pl.BlockSpec((BN, H), lambda i: (i, 0)),
            pl.BlockSpec((BN, HEADS), lambda i: (i, 0)),
            pl.BlockSpec((BN, H), lambda i: (i, 0)),
            pl.BlockSpec((BN, H), lambda i: (i, 0)),
            full((HEADS, H)),
            full((H, H)), full((1, H)), full((1, H)), full((1, H)),
            full((H, 4 * H)), full((1, 4 * H)),
            full((4 * H, H)), full((1, H)), full((1, H)), full((1, H)),
            full((H, 3 * H)), full((1, 3 * H)),
            full((H, 3 * H)), full((1, 3 * H)),
            full((1, H)), full((1, H)),
        ],
        out_specs=[
            pl.BlockSpec((BN, H), lambda i: (i, 0)),
            pl.BlockSpec((BN, H), lambda i: (i, 0)),
        ],
        out_shape=[out, out],
    )(aggr, asum8, x, h, expand, *wp)


def kernel(x, edge_index, edge_attr, params):
    p = params
    src = edge_index[0]
    dst = edge_index[1]

    wqt = p["Wq"].T
    wkt = p["Wk"].T
    wvt = p["Wv"].T
    bq2 = p["bq"].reshape(1, H)
    bk2 = p["bk"].reshape(1, H)
    bv2 = p["bv"].reshape(1, H)

    expand = jnp.repeat(jnp.eye(HEADS, dtype=jnp.float32), DH, axis=1)

    wp = (
        p["Wao"].T, p["bao"].reshape(1, H),
        p["ln1_w"].reshape(1, H), p["ln1_b"].reshape(1, H),
        p["Wi"].T, p["bi"].reshape(1, 4 * H),
        p["Wo"].T, p["bo"].reshape(1, H),
        p["ln2_w"].reshape(1, H), p["ln2_b"].reshape(1, H),
        p["W_ih"].T, p["b_ih"].reshape(1, 3 * H),
        p["W_hh"].T, p["b_hh"].reshape(1, 3 * H),
        p["ln3_w"].reshape(1, H), p["ln3_b"].reshape(1, H),
    )

    ek, ev = _project_edges(edge_attr, wkt, wvt, bk2, bv2)

    expt = expand.T

    h = x
    for _ in range(3):
        qn, kn, vn = _project_nodes(x, wqt, wkt, wvt, bq2)
        qg = jnp.take(qn, dst, axis=0)
        kg = jnp.take(kn, src, axis=0)
        vg = jnp.take(vn, src, axis=0)
        msg, pe = _edge_math(qg, kg, vg, ek, ev, expt, expand)
        aggr = jax.ops.segment_sum(msg, dst, num_segments=N)
        asum8 = jax.ops.segment_sum(pe, dst, num_segments=N)
        x, h = _dense_update(aggr, asum8, x, h, expand, wp)
    return x
